# baseline (speedup 1.0000x reference)
import functools

import jax
import jax.numpy as jnp
from jax import lax
from jax.experimental import pallas as pl
from jax.experimental.pallas import tpu as pltpu
from jax.experimental.pallas import tpu_sc as plsc

N = 10000
D = 128
NC = 2
NS = 16
NW = NC * NS
CHUNK = 80
G = 16
NB = 4
NPAD = N + 112
TROWS = NPAD // NS


def _mesh():
    return plsc.VectorSubcoreMesh(
        core_axis_name="c", subcore_axis_name="s", num_cores=NC, num_subcores=NS
    )


def _deg_body(ch, dstb, ones, zeros, out, dst_v0, dst_v1, ones_v, accd,
              semi0, semi1, sems):
    c = lax.axis_index("c")
    s = lax.axis_index("s")
    w = c * NS + s
    r0 = s * (NPAD // NS)
    ngroups = ch // G
    dvb = (dst_v0, dst_v1)
    semi = (semi0, semi1)
    pltpu.sync_copy(ones.at[...], ones_v)
    pltpu.sync_copy(zeros.at[pl.ds(r0, TROWS)], accd.at[pl.ds(r0, TROWS)])
    plsc.subcore_barrier()

    pltpu.sync_copy(dstb.at[w, pl.ds(0, G)], dvb[0])
    idxd = [None, None]
    pend = [[], []]
    for g in range(ngroups):
        ib = g % 2
        if idxd[ib] is not None:
            idxd[ib].wait()
        nb = (g + 1) % 2
        if g + 1 < ngroups:
            for d in pend[nb]:
                d.wait()
            pend[nb] = []
            idxd[nb] = pltpu.async_copy(
                dstb.at[w, pl.ds((g + 1) * G, G)], dvb[nb], semi[nb]
            )
        for j in range(G):
            pend[ib].append(
                pltpu.async_copy(ones_v, accd.at[dvb[ib].at[j]], sems, add=True)
            )
    for d in pend[0] + pend[1]:
        d.wait()
    plsc.subcore_barrier()
    pltpu.sync_copy(accd.at[pl.ds(r0, TROWS)], out.at[c, pl.ds(r0, TROWS)])


def _deg_call(dstb, ones, zeros, ch):
    k = pl.kernel(
        functools.partial(_deg_body, ch),
        out_type=jax.ShapeDtypeStruct((NC, NPAD, D), jnp.float32),
        mesh=_mesh(),
        scratch_types=[
            pltpu.VMEM((G, CHUNK), jnp.int32),
            pltpu.VMEM((G, CHUNK), jnp.int32),
            pltpu.VMEM((CHUNK, D), jnp.float32),
            pltpu.VMEM_SHARED((NPAD, D), jnp.float32),
            pltpu.SemaphoreType.DMA,
            pltpu.SemaphoreType.DMA,
            pltpu.SemaphoreType.DMA,
        ],
    )
    return k(dstb, ones, zeros)


def _agg_body(ch, y, srcb, dstb, zeros, out, src_v0, src_v1, dst_v0, dst_v1,
              buf0, buf1, buf2, buf3, acc, semg0, semg1, semg2, semg3,
              sems0, sems1, sems2, sems3, semi0, semi1):
    c = lax.axis_index("c")
    s = lax.axis_index("s")
    w = c * NS + s
    r0 = s * (NPAD // NS)
    ngroups = ch // G
    svb = (src_v0, src_v1)
    dvb = (dst_v0, dst_v1)
    bufs = (buf0, buf1, buf2, buf3)
    semg = (semg0, semg1, semg2, semg3)
    sems = (sems0, sems1, sems2, sems3)
    semi = (semi0, semi1)
    pltpu.sync_copy(zeros.at[pl.ds(r0, TROWS)], acc.at[pl.ds(r0, TROWS)])
    plsc.subcore_barrier()

    pltpu.sync_copy(srcb.at[w, pl.ds(0, G)], svb[0])
    pltpu.sync_copy(dstb.at[w, pl.ds(0, G)], dvb[0])
    gd = [None] * NB
    sd = [None] * NB
    idxd = [None, None]

    def scatter(chunk):
        b = chunk % NB
        gd[b].wait()
        dv = dvb[(chunk // G) % 2]
        sd[b] = pltpu.async_copy(
            bufs[b], acc.at[dv.at[chunk % G]], sems[b], add=True
        )

    for g in range(ngroups):
        ib = g % 2
        if idxd[ib] is not None:
            idxd[ib][0].wait()
            idxd[ib][1].wait()
        for j in range(G):
            chunk = g * G + j
            b = chunk % NB
            if sd[b] is not None:
                sd[b].wait()
            gd[b] = pltpu.async_copy(y.at[svb[ib].at[j]], bufs[b], semg[b])
            if chunk >= 1:
                scatter(chunk - 1)
            if j == NB - 1 and g + 1 < ngroups:
                nb = (g + 1) % 2
                idxd[nb] = (
                    pltpu.async_copy(
                        srcb.at[w, pl.ds((g + 1) * G, G)], svb[nb], semi[nb]
                    ),
                    pltpu.async_copy(
                        dstb.at[w, pl.ds((g + 1) * G, G)], dvb[nb], semi[nb]
                    ),
                )
    scatter(ngroups * G - 1)
    for d in sd:
        if d is not None:
            d.wait()
    plsc.subcore_barrier()
    pltpu.sync_copy(acc.at[pl.ds(r0, TROWS)], out.at[c, pl.ds(r0, TROWS)])


def _agg_call(y, srcb, dstb, zeros, ch):
    k = pl.kernel(
        functools.partial(_agg_body, ch),
        out_type=jax.ShapeDtypeStruct((NC, NPAD, D), jnp.float32),
        mesh=_mesh(),
        scratch_types=[
            pltpu.VMEM((G, CHUNK), jnp.int32),
            pltpu.VMEM((G, CHUNK), jnp.int32),
            pltpu.VMEM((G, CHUNK), jnp.int32),
            pltpu.VMEM((G, CHUNK), jnp.int32),
            pltpu.VMEM((CHUNK, D), jnp.float32),
            pltpu.VMEM((CHUNK, D), jnp.float32),
            pltpu.VMEM((CHUNK, D), jnp.float32),
            pltpu.VMEM((CHUNK, D), jnp.float32),
            pltpu.VMEM_SHARED((NPAD, D), jnp.float32),
            pltpu.SemaphoreType.DMA,
            pltpu.SemaphoreType.DMA,
            pltpu.SemaphoreType.DMA,
            pltpu.SemaphoreType.DMA,
            pltpu.SemaphoreType.DMA,
            pltpu.SemaphoreType.DMA,
            pltpu.SemaphoreType.DMA,
            pltpu.SemaphoreType.DMA,
            pltpu.SemaphoreType.DMA,
            pltpu.SemaphoreType.DMA,
        ],
    )
    return k(y, srcb, dstb, zeros)


_BN = 1000


def _prep_body(x_ref, degp_ref, y_ref, dis_ref):
    deg = degp_ref[0][:, 0:8] + degp_ref[1][:, 0:8] + 1.0
    disv = lax.rsqrt(deg)
    dis_ref[...] = disv
    y_ref[...] = x_ref[...] * disv[:, 0:1]


def _prep_call(x, degp):
    return pl.pallas_call(
        _prep_body,
        grid=(N // _BN,),
        in_specs=[
            pl.BlockSpec((_BN, D), lambda i: (i, 0)),
            pl.BlockSpec((NC, _BN, D), lambda i: (0, i, 0)),
        ],
        out_specs=[
            pl.BlockSpec((_BN, D), lambda i: (i, 0)),
            pl.BlockSpec((_BN, 8), lambda i: (i, 0)),
        ],
        out_shape=[
            jax.ShapeDtypeStruct((N, D), jnp.float32),
            jax.ShapeDtypeStruct((N, 8), jnp.float32),
        ],
    )(x, degp)


def _layer_body(last, z_ref, y_ref, dis_ref, w_ref, b_ref, out_ref):
    dis = dis_ref[:, 0:1]
    agg = (z_ref[0] + z_ref[1] + y_ref[...]) * dis
    h = jnp.dot(agg, w_ref[...], preferred_element_type=jnp.float32)
    h = h + b_ref[...]
    if last:
        out_ref[...] = h
    else:
        out_ref[...] = jnp.maximum(h, 0.0) * dis


def _layer_call(z, y, dis, w, b, last):
    return pl.pallas_call(
        functools.partial(_layer_body, last),
        grid=(N // _BN,),
        in_specs=[
            pl.BlockSpec((NC, _BN, D), lambda i: (0, i, 0)),
            pl.BlockSpec((_BN, D), lambda i: (i, 0)),
            pl.BlockSpec((_BN, 8), lambda i: (i, 0)),
            pl.BlockSpec((D, D), lambda i: (0, 0)),
            pl.BlockSpec((1, D), lambda i: (0, 0)),
        ],
        out_specs=pl.BlockSpec((_BN, D), lambda i: (i, 0)),
        out_shape=jax.ShapeDtypeStruct((N, D), jnp.float32),
    )(z, y, dis, w, b.reshape(1, D))


def kernel(x, edge_index, W1, b1, W2, b2, W3, b3):
    src = edge_index[0].astype(jnp.int32)
    dst = edge_index[1].astype(jnp.int32)
    e = src.shape[0]
    per_w = pl.cdiv(pl.cdiv(e, NW), G * CHUNK) * G * CHUNK
    ch = per_w // CHUNK
    tot = per_w * NW
    pad = tot - e
    pad_ix = jnp.arange(pad, dtype=jnp.int32)
    src_p = jnp.concatenate([src, (pad_ix * 37) % N])
    dst_p = jnp.concatenate([dst, N + (pad_ix % (NPAD - N))])
    srcb = src_p.reshape(NW, ch, CHUNK)
    dstb = dst_p.reshape(NW, ch, CHUNK)

    ones = jnp.ones((CHUNK, D), jnp.float32)
    zeros = jnp.zeros((NPAD, D), jnp.float32)

    degp = _deg_call(dstb, ones, zeros, ch)
    y1, dis = _prep_call(x, degp)
    z1 = _agg_call(y1, srcb, dstb, zeros, ch)
    y2 = _layer_call(z1, y1, dis, W1, b1, last=False)
    z2 = _agg_call(y2, srcb, dstb, zeros, ch)
    y3 = _layer_call(z2, y2, dis, W2, b2, last=False)
    z3 = _agg_call(y3, srcb, dstb, zeros, ch)
    h = _layer_call(z3, y3, dis, W3, b3, last=True)
    return (h, h)


# speedup vs baseline: 1.0556x; 1.0556x over previous
import functools

import jax
import jax.numpy as jnp
from jax import lax
from jax.experimental import pallas as pl
from jax.experimental.pallas import tpu as pltpu
from jax.experimental.pallas import tpu_sc as plsc

N = 10000
D = 128
NC = 2
NS = 16
NW = NC * NS
CHUNK = 128
G = 16
NB = 2
NPAD = N + 112
TROWS = NPAD // NS


def _mesh():
    return plsc.VectorSubcoreMesh(
        core_axis_name="c", subcore_axis_name="s", num_cores=NC, num_subcores=NS
    )


def _deg_body(ch, dstb, ones, zeros, out, dst_v0, dst_v1, ones_v, accd,
              semi0, semi1, sems):
    c = lax.axis_index("c")
    s = lax.axis_index("s")
    w = c * NS + s
    r0 = s * (NPAD // NS)
    ngroups = ch // G
    dvb = (dst_v0, dst_v1)
    semi = (semi0, semi1)
    pltpu.sync_copy(ones.at[...], ones_v)
    pltpu.sync_copy(zeros.at[pl.ds(r0, TROWS)], accd.at[pl.ds(r0, TROWS)])
    plsc.subcore_barrier()

    pltpu.sync_copy(dstb.at[w, pl.ds(0, G)], dvb[0])
    idxd = [None, None]
    pend = [[], []]
    for g in range(ngroups):
        ib = g % 2
        if idxd[ib] is not None:
            idxd[ib].wait()
        nb = (g + 1) % 2
        if g + 1 < ngroups:
            for d in pend[nb]:
                d.wait()
            pend[nb] = []
            idxd[nb] = pltpu.async_copy(
                dstb.at[w, pl.ds((g + 1) * G, G)], dvb[nb], semi[nb]
            )
        for j in range(G):
            pend[ib].append(
                pltpu.async_copy(ones_v, accd.at[dvb[ib].at[j]], sems, add=True)
            )
    for d in pend[0] + pend[1]:
        d.wait()
    plsc.subcore_barrier()
    pltpu.sync_copy(accd.at[pl.ds(r0, TROWS)], out.at[c, pl.ds(r0, TROWS)])


def _deg_call(dstb, ones, zeros, ch):
    k = pl.kernel(
        functools.partial(_deg_body, ch),
        out_type=jax.ShapeDtypeStruct((NC, NPAD, D), jnp.float32),
        mesh=_mesh(),
        scratch_types=[
            pltpu.VMEM((G, CHUNK), jnp.int32),
            pltpu.VMEM((G, CHUNK), jnp.int32),
            pltpu.VMEM((CHUNK, D), jnp.float32),
            pltpu.VMEM_SHARED((NPAD, D), jnp.float32),
            pltpu.SemaphoreType.DMA,
            pltpu.SemaphoreType.DMA,
            pltpu.SemaphoreType.DMA,
        ],
    )
    return k(dstb, ones, zeros)


def _agg_body(ch, y, srcb, dstb, zeros, out, src_v0, src_v1, dst_v0, dst_v1,
              buf0, buf1, acc, semg0, semg1, sems0, sems1, semi0, semi1):
    c = lax.axis_index("c")
    s = lax.axis_index("s")
    w = c * NS + s
    r0 = s * (NPAD // NS)
    ngroups = ch // G
    svb = (src_v0, src_v1)
    dvb = (dst_v0, dst_v1)
    bufs = (buf0, buf1)
    semg = (semg0, semg1)
    sems = (sems0, sems1)
    semi = (semi0, semi1)
    pltpu.sync_copy(zeros.at[pl.ds(r0, TROWS)], acc.at[pl.ds(r0, TROWS)])
    plsc.subcore_barrier()

    pltpu.sync_copy(srcb.at[w, pl.ds(0, G)], svb[0])
    pltpu.sync_copy(dstb.at[w, pl.ds(0, G)], dvb[0])
    gd = [None] * NB
    sd = [None] * NB
    idxd = [None, None]

    def scatter(chunk):
        b = chunk % NB
        gd[b].wait()
        dv = dvb[(chunk // G) % 2]
        sd[b] = pltpu.async_copy(
            bufs[b], acc.at[dv.at[chunk % G]], sems[b], add=True
        )

    for g in range(ngroups):
        ib = g % 2
        if idxd[ib] is not None:
            idxd[ib][0].wait()
            idxd[ib][1].wait()
        for j in range(G):
            chunk = g * G + j
            b = chunk % NB
            if sd[b] is not None:
                sd[b].wait()
            gd[b] = pltpu.async_copy(y.at[svb[ib].at[j]], bufs[b], semg[b])
            if chunk >= 1:
                scatter(chunk - 1)
            if j == NB - 1 and g + 1 < ngroups:
                nb = (g + 1) % 2
                idxd[nb] = (
                    pltpu.async_copy(
                        srcb.at[w, pl.ds((g + 1) * G, G)], svb[nb], semi[nb]
                    ),
                    pltpu.async_copy(
                        dstb.at[w, pl.ds((g + 1) * G, G)], dvb[nb], semi[nb]
                    ),
                )
    scatter(ngroups * G - 1)
    for d in sd:
        if d is not None:
            d.wait()
    plsc.subcore_barrier()
    pltpu.sync_copy(acc.at[pl.ds(r0, TROWS)], out.at[c, pl.ds(r0, TROWS)])


def _agg_call(y, srcb, dstb, zeros, ch):
    k = pl.kernel(
        functools.partial(_agg_body, ch),
        out_type=jax.ShapeDtypeStruct((NC, NPAD, D), jnp.float32),
        mesh=_mesh(),
        scratch_types=[
            pltpu.VMEM((G, CHUNK), jnp.int32),
            pltpu.VMEM((G, CHUNK), jnp.int32),
            pltpu.VMEM((G, CHUNK), jnp.int32),
            pltpu.VMEM((G, CHUNK), jnp.int32),
            pltpu.VMEM((CHUNK, D), jnp.float32),
            pltpu.VMEM((CHUNK, D), jnp.float32),
            pltpu.VMEM_SHARED((NPAD, D), jnp.float32),
            pltpu.SemaphoreType.DMA,
            pltpu.SemaphoreType.DMA,
            pltpu.SemaphoreType.DMA,
            pltpu.SemaphoreType.DMA,
            pltpu.SemaphoreType.DMA,
            pltpu.SemaphoreType.DMA,
        ],
    )
    return k(y, srcb, dstb, zeros)


_BN = 1000


def _prep_body(x_ref, degp_ref, y_ref, dis_ref):
    deg = degp_ref[0][:, 0:8] + degp_ref[1][:, 0:8] + 1.0
    disv = lax.rsqrt(deg)
    dis_ref[...] = disv
    y_ref[...] = x_ref[...] * disv[:, 0:1]


def _prep_call(x, degp):
    return pl.pallas_call(
        _prep_body,
        grid=(N // _BN,),
        in_specs=[
            pl.BlockSpec((_BN, D), lambda i: (i, 0)),
            pl.BlockSpec((NC, _BN, D), lambda i: (0, i, 0)),
        ],
        out_specs=[
            pl.BlockSpec((_BN, D), lambda i: (i, 0)),
            pl.BlockSpec((_BN, 8), lambda i: (i, 0)),
        ],
        out_shape=[
            jax.ShapeDtypeStruct((N, D), jnp.float32),
            jax.ShapeDtypeStruct((N, 8), jnp.float32),
        ],
    )(x, degp)


def _layer_body(last, z_ref, y_ref, dis_ref, w_ref, b_ref, out_ref):
    dis = dis_ref[:, 0:1]
    agg = (z_ref[0] + z_ref[1] + y_ref[...]) * dis
    h = jnp.dot(agg, w_ref[...], preferred_element_type=jnp.float32)
    h = h + b_ref[...]
    if last:
        out_ref[...] = h
    else:
        out_ref[...] = jnp.maximum(h, 0.0) * dis


def _layer_call(z, y, dis, w, b, last):
    return pl.pallas_call(
        functools.partial(_layer_body, last),
        grid=(N // _BN,),
        in_specs=[
            pl.BlockSpec((NC, _BN, D), lambda i: (0, i, 0)),
            pl.BlockSpec((_BN, D), lambda i: (i, 0)),
            pl.BlockSpec((_BN, 8), lambda i: (i, 0)),
            pl.BlockSpec((D, D), lambda i: (0, 0)),
            pl.BlockSpec((1, D), lambda i: (0, 0)),
        ],
        out_specs=pl.BlockSpec((_BN, D), lambda i: (i, 0)),
        out_shape=jax.ShapeDtypeStruct((N, D), jnp.float32),
    )(z, y, dis, w, b.reshape(1, D))


def kernel(x, edge_index, W1, b1, W2, b2, W3, b3):
    src = edge_index[0].astype(jnp.int32)
    dst = edge_index[1].astype(jnp.int32)
    e = src.shape[0]
    per_w = pl.cdiv(pl.cdiv(e, NW), G * CHUNK) * G * CHUNK
    ch = per_w // CHUNK
    tot = per_w * NW
    pad = tot - e
    pad_ix = jnp.arange(pad, dtype=jnp.int32)
    src_p = jnp.concatenate([src, (pad_ix * 37) % N])
    dst_p = jnp.concatenate([dst, N + (pad_ix % (NPAD - N))])
    srcb = src_p.reshape(NW, ch, CHUNK)
    dstb = dst_p.reshape(NW, ch, CHUNK)

    ones = jnp.ones((CHUNK, D), jnp.float32)
    zeros = jnp.zeros((NPAD, D), jnp.float32)

    degp = _deg_call(dstb, ones, zeros, ch)
    y1, dis = _prep_call(x, degp)
    z1 = _agg_call(y1, srcb, dstb, zeros, ch)
    y2 = _layer_call(z1, y1, dis, W1, b1, last=False)
    z2 = _agg_call(y2, srcb, dstb, zeros, ch)
    y3 = _layer_call(z2, y2, dis, W2, b2, last=False)
    z3 = _agg_call(y3, srcb, dstb, zeros, ch)
    h = _layer_call(z3, y3, dis, W3, b3, last=True)
    return (h, h)


# speedup vs baseline: 1.0709x; 1.0146x over previous
import functools

import jax
import jax.numpy as jnp
from jax import lax
from jax.experimental import pallas as pl
from jax.experimental.pallas import tpu as pltpu
from jax.experimental.pallas import tpu_sc as plsc

N = 10000
D = 128
NC = 2
NS = 16
NW = NC * NS
CHUNK = 128
G = 16
NB = 2
NPAD = N + 112
TROWS = NPAD // NS


def _mesh():
    return plsc.VectorSubcoreMesh(
        core_axis_name="c", subcore_axis_name="s", num_cores=NC, num_subcores=NS
    )


def _deg_body(ch, dstb, ones, zeros, out, dst_v0, dst_v1, ones_v, accd,
              semi0, semi1, sems):
    c = lax.axis_index("c")
    s = lax.axis_index("s")
    w = c * NS + s
    r0 = s * (NPAD // NS)
    ngroups = ch // G
    dvb = (dst_v0, dst_v1)
    semi = (semi0, semi1)
    pltpu.sync_copy(ones.at[...], ones_v)
    pltpu.sync_copy(zeros.at[pl.ds(r0, TROWS)], accd.at[pl.ds(r0, TROWS)])
    plsc.subcore_barrier()

    pltpu.sync_copy(dstb.at[w, pl.ds(0, G)], dvb[0])
    idxd = [None, None]
    pend = [[], []]
    for g in range(ngroups):
        ib = g % 2
        if idxd[ib] is not None:
            idxd[ib].wait()
        nb = (g + 1) % 2
        if g + 1 < ngroups:
            for d in pend[nb]:
                d.wait()
            pend[nb] = []
            idxd[nb] = pltpu.async_copy(
                dstb.at[w, pl.ds((g + 1) * G, G)], dvb[nb], semi[nb]
            )
        for j in range(G):
            pend[ib].append(
                pltpu.async_copy(ones_v, accd.at[dvb[ib].at[j]], sems, add=True)
            )
    for d in pend[0] + pend[1]:
        d.wait()
    plsc.subcore_barrier()
    pltpu.sync_copy(accd.at[pl.ds(r0, TROWS)], out.at[c, pl.ds(r0, TROWS)])


def _deg_call(dstb, ones, zeros, ch):
    k = pl.kernel(
        functools.partial(_deg_body, ch),
        out_type=jax.ShapeDtypeStruct((NC, NPAD, D), jnp.float32),
        mesh=_mesh(),
        scratch_types=[
            pltpu.VMEM((G, CHUNK), jnp.int32),
            pltpu.VMEM((G, CHUNK), jnp.int32),
            pltpu.VMEM((CHUNK, D), jnp.float32),
            pltpu.VMEM_SHARED((NPAD, D), jnp.float32),
            pltpu.SemaphoreType.DMA,
            pltpu.SemaphoreType.DMA,
            pltpu.SemaphoreType.DMA,
        ],
    )
    return k(dstb, ones, zeros)


def _agg_body(ch, y, srcb, dstb, zeros, out, src_v0, src_v1, dst_v0, dst_v1,
              buf0, buf1, acc, semg0, semg1, sems0, sems1, semi0, semi1):
    c = lax.axis_index("c")
    s = lax.axis_index("s")
    w = c * NS + s
    r0 = s * (NPAD // NS)
    ngroups = ch // G
    svb = (src_v0, src_v1)
    dvb = (dst_v0, dst_v1)
    bufs = (buf0, buf1)
    semg = (semg0, semg1)
    sems = (sems0, sems1)
    semi = (semi0, semi1)
    pltpu.sync_copy(zeros.at[pl.ds(r0, TROWS)], acc.at[pl.ds(r0, TROWS)])
    plsc.subcore_barrier()

    pltpu.sync_copy(srcb.at[w, pl.ds(0, G)], svb[0])
    pltpu.sync_copy(dstb.at[w, pl.ds(0, G)], dvb[0])
    gd = [None] * NB
    sd = [None] * NB
    idxd = [None, None]

    def scatter(chunk):
        b = chunk % NB
        gd[b].wait()
        dv = dvb[(chunk // G) % 2]
        sd[b] = pltpu.async_copy(
            bufs[b], acc.at[dv.at[chunk % G]], sems[b], add=True
        )

    for g in range(ngroups):
        ib = g % 2
        if idxd[ib] is not None:
            idxd[ib][0].wait()
            idxd[ib][1].wait()
        for j in range(G):
            chunk = g * G + j
            b = chunk % NB
            if sd[b] is not None:
                sd[b].wait()
            gd[b] = pltpu.async_copy(y.at[svb[ib].at[j]], bufs[b], semg[b])
            if chunk >= 1:
                scatter(chunk - 1)
            if j == NB - 1 and g + 1 < ngroups:
                nb = (g + 1) % 2
                idxd[nb] = (
                    pltpu.async_copy(
                        srcb.at[w, pl.ds((g + 1) * G, G)], svb[nb], semi[nb]
                    ),
                    pltpu.async_copy(
                        dstb.at[w, pl.ds((g + 1) * G, G)], dvb[nb], semi[nb]
                    ),
                )
    scatter(ngroups * G - 1)
    for d in sd:
        if d is not None:
            d.wait()
    plsc.subcore_barrier()
    pltpu.sync_copy(acc.at[pl.ds(r0, TROWS)], out.at[c, pl.ds(r0, TROWS)])


def _agg_call(y, srcb, dstb, zeros, ch):
    k = pl.kernel(
        functools.partial(_agg_body, ch),
        out_type=jax.ShapeDtypeStruct((NC, NPAD, D), jnp.float32),
        mesh=_mesh(),
        scratch_types=[
            pltpu.VMEM((G, CHUNK), jnp.int32),
            pltpu.VMEM((G, CHUNK), jnp.int32),
            pltpu.VMEM((G, CHUNK), jnp.int32),
            pltpu.VMEM((G, CHUNK), jnp.int32),
            pltpu.VMEM((CHUNK, D), jnp.float32),
            pltpu.VMEM((CHUNK, D), jnp.float32),
            pltpu.VMEM_SHARED((NPAD, D), jnp.float32),
            pltpu.SemaphoreType.DMA,
            pltpu.SemaphoreType.DMA,
            pltpu.SemaphoreType.DMA,
            pltpu.SemaphoreType.DMA,
            pltpu.SemaphoreType.DMA,
            pltpu.SemaphoreType.DMA,
        ],
    )
    return k(y, srcb, dstb, zeros)


_BN = 2000


def _prep_body(x_ref, degp_ref, y_ref, dis_ref):
    deg = degp_ref[0][:, 0:8] + degp_ref[1][:, 0:8] + 1.0
    disv = lax.rsqrt(deg)
    dis_ref[...] = disv
    y_ref[...] = x_ref[...] * disv[:, 0:1]


def _prep_call(x, degp):
    return pl.pallas_call(
        _prep_body,
        grid=(N // _BN,),
        in_specs=[
            pl.BlockSpec((_BN, D), lambda i: (i, 0)),
            pl.BlockSpec((NC, _BN, D), lambda i: (0, i, 0)),
        ],
        out_specs=[
            pl.BlockSpec((_BN, D), lambda i: (i, 0)),
            pl.BlockSpec((_BN, 8), lambda i: (i, 0)),
        ],
        out_shape=[
            jax.ShapeDtypeStruct((N, D), jnp.float32),
            jax.ShapeDtypeStruct((N, 8), jnp.float32),
        ],
    )(x, degp)


def _layer_body(last, z_ref, y_ref, dis_ref, w_ref, b_ref, out_ref):
    dis = dis_ref[:, 0:1]
    agg = (z_ref[0] + z_ref[1] + y_ref[...]) * dis
    h = jnp.dot(agg, w_ref[...], preferred_element_type=jnp.float32)
    h = h + b_ref[...]
    if last:
        out_ref[...] = h
    else:
        out_ref[...] = jnp.maximum(h, 0.0) * dis


def _layer_call(z, y, dis, w, b, last):
    return pl.pallas_call(
        functools.partial(_layer_body, last),
        grid=(N // _BN,),
        in_specs=[
            pl.BlockSpec((NC, _BN, D), lambda i: (0, i, 0)),
            pl.BlockSpec((_BN, D), lambda i: (i, 0)),
            pl.BlockSpec((_BN, 8), lambda i: (i, 0)),
            pl.BlockSpec((D, D), lambda i: (0, 0)),
            pl.BlockSpec((1, D), lambda i: (0, 0)),
        ],
        out_specs=pl.BlockSpec((_BN, D), lambda i: (i, 0)),
        out_shape=jax.ShapeDtypeStruct((N, D), jnp.float32),
    )(z, y, dis, w, b.reshape(1, D))


def kernel(x, edge_index, W1, b1, W2, b2, W3, b3):
    src = edge_index[0].astype(jnp.int32)
    dst = edge_index[1].astype(jnp.int32)
    e = src.shape[0]
    per_w = pl.cdiv(pl.cdiv(e, NW), G * CHUNK) * G * CHUNK
    ch = per_w // CHUNK
    tot = per_w * NW
    pad = tot - e
    pad_ix = jnp.arange(pad, dtype=jnp.int32)
    src_p = jnp.concatenate([src, (pad_ix * 37) % N])
    dst_p = jnp.concatenate([dst, N + (pad_ix % (NPAD - N))])
    srcb = src_p.reshape(NW, ch, CHUNK)
    dstb = dst_p.reshape(NW, ch, CHUNK)

    ones = jnp.ones((CHUNK, D), jnp.float32)
    zeros = jnp.zeros((NPAD, D), jnp.float32)

    degp = _deg_call(dstb, ones, zeros, ch)
    y1, dis = _prep_call(x, degp)
    z1 = _agg_call(y1, srcb, dstb, zeros, ch)
    y2 = _layer_call(z1, y1, dis, W1, b1, last=False)
    z2 = _agg_call(y2, srcb, dstb, zeros, ch)
    y3 = _layer_call(z2, y2, dis, W2, b2, last=False)
    z3 = _agg_call(y3, srcb, dstb, zeros, ch)
    h = _layer_call(z3, y3, dis, W3, b3, last=True)
    return (h, h)
